# baseline (speedup 1.0000x reference)
import functools

import jax
import jax.numpy as jnp
from jax import lax
from jax.experimental import pallas as pl
from jax.experimental.pallas import tpu as pltpu
from jax.experimental.pallas import tpu_sc as plsc

N = 100000
B = 8
D = 128
NPAD = 100352
TN = 2048
TC_GRID = NPAD // TN

NT = 16
NC = 2
RADIX = 256
NPASS = 4
CHUNK = NPAD // NT
VREGS = CHUNK // 16
SLICES = CHUNK // 128
ROWS_PER_CORE = B // NC


def _score_keys_body(ent_ref, vt_ref, wt_ref, fb_ref, out_ref, *, b0, nb):
    i = pl.program_id(0)
    et = jnp.transpose(ent_ref[...])
    rows = []
    for bl in range(nb):
        b = b0 + bl
        diff = jnp.abs(vt_ref[:, b:b + 1] - wt_ref[:, b:b + 1] * et)
        a = diff[0:8, :]
        for k in range(1, 16):
            a = a + diff[8 * k:8 * k + 8, :]
        for sh in (4, 2, 1):
            a = jnp.concatenate([a[sh:, :], a[:sh, :]], axis=0) + a
        rows.append(a[b:b + 1, :])
    score = jnp.concatenate(rows, axis=0) + fb_ref[...]
    bits = lax.bitcast_convert_type(score, jnp.int32)
    key = jnp.where(bits < 0, ~bits, bits ^ jnp.int32(-2147483648))
    col = i * TN + lax.broadcasted_iota(jnp.int32, (nb, TN), 1)
    out_ref[...] = jnp.where(col < N, key, jnp.int32(-1))


def _score_keys(ent_embd, vt, wt, fb_part, b0, nb):
    return pl.pallas_call(
        functools.partial(_score_keys_body, b0=b0, nb=nb),
        grid=(TC_GRID,),
        in_specs=[
            pl.BlockSpec((TN, D), lambda i: (i, 0)),
            pl.BlockSpec((D, B), lambda i: (0, 0)),
            pl.BlockSpec((D, B), lambda i: (0, 0)),
            pl.BlockSpec((nb, TN), lambda i: (0, i)),
        ],
        out_specs=pl.BlockSpec((nb, TN), lambda i: (0, i)),
        out_shape=jax.ShapeDtypeStruct((nb, NPAD), jnp.int32),
    )(ent_embd, vt, wt, fb_part)


def _radix_body(keys_hbm, out_hbm, kv, vv, vinit, dbuf, cntbuf, lastbuf,
                posv, hb, hba, hm, hist_sp, ka, va, kb, vb, sem, *,
                rows_per_core):
    c = lax.axis_index("c")
    t = lax.axis_index("s")
    lane = lax.iota(jnp.int32, 16)
    cbase = t * CHUNK

    def digit_of(k, sh):
        if sh:
            k = lax.shift_right_logical(k, jnp.int32(sh))
        return lax.bitwise_and(k, jnp.int32(RADIX - 1))

    def gloop(j, _):
        vinit[pl.ds(j * 16, 16)] = cbase + j * 16 + lane
        return 0
    lax.fori_loop(0, VREGS, gloop, 0)

    HALF_A = 24

    def one_pass(sh, src_v, dst_k, dst_v):
        def zloop(z):
            hb[pl.ds(z * 16, 16)] = jnp.zeros((16,), jnp.int32)
        plsc.parallel_loop(0, RADIX // 16, unroll=4)(zloop)

        def hloop(j):
            sl = pl.ds(j * 16, 16)
            d = digit_of(kv[sl], sh)
            cnt, lastm = plsc.scan_count(d)
            dbuf[sl] = d
            cntbuf[sl] = cnt
            lastbuf[sl] = jnp.where(lastm, 1, 0).astype(jnp.int32)
            plsc.addupdate_scatter(hb, [d], cnt, mask=lastm)
        plsc.parallel_loop(0, HALF_A * 8, unroll=4)(hloop)

        def cloop(z):
            hba[pl.ds(z * 16, 16)] = hb[pl.ds(z * 16, 16)]
        plsc.parallel_loop(0, RADIX // 16, unroll=4)(cloop)
        plsc.parallel_loop(HALF_A * 8, VREGS, unroll=4)(hloop)
        pltpu.sync_copy(hb, hist_sp.at[t])
        plsc.subcore_barrier()

        pltpu.sync_copy(hist_sp, hm)

        def sloop(dv, carry):
            sl = pl.ds(dv * 16, 16)
            z16 = jnp.zeros((16,), jnp.int32)

            def tloop(tp, tc):
                tot, part = tc
                hv = hm[tp, sl]
                tot = tot + hv
                part = part + jnp.where(tp < t, hv, z16)
                return tot, part
            tot, part = lax.fori_loop(0, NT, tloop, (z16, z16))
            incl = plsc.cumsum(tot)
            hb[sl] = incl - tot + part + carry
            return carry + jnp.max(incl)
        lax.fori_loop(0, RADIX // 16, sloop, jnp.int32(0))

        def bloop(z, _):
            sl = pl.ds(z * 16, 16)
            hba[sl] = hba[sl] + hb[sl]
            return 0
        lax.fori_loop(0, RADIX // 16, bloop, 0)

        n_dma = 2 if dst_k is not None else 1

        def rank_vreg(j, base_ref):
            sl = pl.ds(j * 16, 16)
            d = dbuf[sl]
            cnt = cntbuf[sl]
            lastm = lastbuf[sl] != 0
            base = plsc.load_gather(base_ref, [d])
            posv[j >> 3, pl.ds((j & 7) * 16, 16)] = base + cnt - 1
            plsc.addupdate_scatter(base_ref, [d], cnt, mask=lastm)

        def fire(s):
            idx = posv.at[s]
            src = pl.ds(s * 128, 128)
            if dst_k is not None:
                pltpu.async_copy(kv.at[src], dst_k.at[idx], sem)
            pltpu.async_copy(src_v.at[src], dst_v.at[idx], sem)

        def rgroup(g, _):
            for q in range(8):
                rank_vreg(g * 8 + q, hb)
                rank_vreg(HALF_A * 8 + g * 8 + q, hba)
            fire(g)
            fire(HALF_A + g)
            return 0
        lax.fori_loop(0, HALF_A, rgroup, 0)
        for q in range(8):
            rank_vreg(2 * HALF_A * 8 + q, hba)
        fire(2 * HALF_A)

        def dloop(s, _):
            pltpu.make_async_copy(keys_hbm.at[pl.ds(0, 128)],
                                  kv.at[pl.ds(0, 128)], sem).wait()
            return 0
        lax.fori_loop(0, SLICES * n_dma, dloop, 0)
        plsc.subcore_barrier()

    def do_row(ri, _):
        row = c * rows_per_core + ri
        hoff = row * NPAD + cbase

        pltpu.sync_copy(keys_hbm.at[pl.ds(hoff, CHUNK)], kv)
        one_pass(0, vinit, ka, va)

        pltpu.sync_copy(ka.at[pl.ds(cbase, CHUNK)], kv)
        pltpu.sync_copy(va.at[pl.ds(cbase, CHUNK)], vv)
        one_pass(8, vv, kb, vb)

        pltpu.sync_copy(kb.at[pl.ds(cbase, CHUNK)], kv)
        pltpu.sync_copy(vb.at[pl.ds(cbase, CHUNK)], vv)
        one_pass(16, vv, ka, va)

        pltpu.sync_copy(ka.at[pl.ds(cbase, CHUNK)], kv)
        pltpu.sync_copy(va.at[pl.ds(cbase, CHUNK)], vv)
        one_pass(24, vv, None, vb)

        pltpu.sync_copy(vb.at[pl.ds(cbase, CHUNK)], out_hbm.at[pl.ds(hoff, CHUNK)])
        plsc.subcore_barrier()
        return 0

    lax.fori_loop(0, rows_per_core, do_row, 0)


@jax.jit
def kernel(sample, filter_bias, ent_embd, rel_embd, wrh, wrt):
    h = jnp.take(ent_embd, sample[:, 0], axis=0)
    r = jnp.take(rel_embd, sample[:, 1], axis=0)
    wh = jnp.take(wrh, sample[:, 1], axis=0)
    wt = jnp.take(wrt, sample[:, 1], axis=0)
    v = wh * h + r

    vt, wtt = v.T, wt.T
    nsplit = 4
    hb2 = B // nsplit
    keys_parts = [
        _score_keys(ent_embd, vt, wtt,
                    filter_bias[p * hb2:(p + 1) * hb2], p * hb2, hb2)
        for p in range(nsplit)
    ]

    mesh = plsc.VectorSubcoreMesh(core_axis_name="c", subcore_axis_name="s")
    radix = functools.partial(
        pl.kernel,
        out_type=jax.ShapeDtypeStruct((hb2 * NPAD,), jnp.int32),
        mesh=mesh,
        compiler_params=pltpu.CompilerParams(needs_layout_passes=False),
        scratch_types=[
            pltpu.VMEM((CHUNK,), jnp.int32),
            pltpu.VMEM((CHUNK,), jnp.int32),
            pltpu.VMEM((CHUNK,), jnp.int32),
            pltpu.VMEM((CHUNK,), jnp.int32),
            pltpu.VMEM((CHUNK,), jnp.int32),
            pltpu.VMEM((CHUNK,), jnp.int32),
            pltpu.VMEM((SLICES, 128), jnp.int32),
            pltpu.VMEM((RADIX,), jnp.int32),
            pltpu.VMEM((RADIX,), jnp.int32),
            pltpu.VMEM((NT, RADIX), jnp.int32),
            pltpu.VMEM_SHARED((NT, RADIX), jnp.int32),
            pltpu.VMEM_SHARED((NPAD,), jnp.int32),
            pltpu.VMEM_SHARED((NPAD,), jnp.int32),
            pltpu.VMEM_SHARED((NPAD,), jnp.int32),
            pltpu.VMEM_SHARED((NPAD,), jnp.int32),
            pltpu.SemaphoreType.DMA,
        ],
    )(functools.partial(_radix_body, rows_per_core=hb2 // NC))
    order = jnp.concatenate(
        [radix(kp.reshape(-1)).reshape(hb2, NPAD) for kp in keys_parts],
        axis=0)
    return order[:, :N]


# speedup vs baseline: 1.1777x; 1.1777x over previous
import functools

import jax
import jax.numpy as jnp
from jax import lax
from jax.experimental import pallas as pl
from jax.experimental.pallas import tpu as pltpu
from jax.experimental.pallas import tpu_sc as plsc

N = 100000
B = 8
D = 128
NPAD = 100352
TN = 2048
TC_GRID = NPAD // TN

NT = 16
NC = 2
RADIX = 256
NPASS = 4
CHUNK = NPAD // NT
VREGS = CHUNK // 16
SLICES = CHUNK // 128
ROWS_PER_CORE = B // NC


def _score_keys_body(ent_ref, vt_ref, wt_ref, fb_ref, out_ref, *, b0, nb):
    i = pl.program_id(0)
    et = jnp.transpose(ent_ref[...])
    rows = []
    for bl in range(nb):
        b = b0 + bl
        diff = jnp.abs(vt_ref[:, b:b + 1] - wt_ref[:, b:b + 1] * et)
        a = diff[0:8, :]
        for k in range(1, 16):
            a = a + diff[8 * k:8 * k + 8, :]
        for sh in (4, 2, 1):
            a = jnp.concatenate([a[sh:, :], a[:sh, :]], axis=0) + a
        rows.append(a[b:b + 1, :])
    score = jnp.concatenate(rows, axis=0) + fb_ref[...]
    bits = lax.bitcast_convert_type(score, jnp.int32)
    key = jnp.where(bits < 0, ~bits, bits ^ jnp.int32(-2147483648))
    col = i * TN + lax.broadcasted_iota(jnp.int32, (nb, TN), 1)
    out_ref[...] = jnp.where(col < N, key, jnp.int32(-1))


def _score_keys(ent_embd, vt, wt, fb_part, b0, nb):
    return pl.pallas_call(
        functools.partial(_score_keys_body, b0=b0, nb=nb),
        grid=(TC_GRID,),
        in_specs=[
            pl.BlockSpec((TN, D), lambda i: (i, 0)),
            pl.BlockSpec((D, B), lambda i: (0, 0)),
            pl.BlockSpec((D, B), lambda i: (0, 0)),
            pl.BlockSpec((nb, TN), lambda i: (0, i)),
        ],
        out_specs=pl.BlockSpec((nb, TN), lambda i: (0, i)),
        out_shape=jax.ShapeDtypeStruct((nb, NPAD), jnp.int32),
    )(ent_embd, vt, wt, fb_part)


def _radix_body(keys_hbm, out_hbm, kv, vv, vinit, dbuf, cntbuf, lastbuf,
                posv, hb, hba, hm, hist_sp, ka, va, kb, vb, sem, *,
                rows_per_core):
    c = lax.axis_index("c")
    t = lax.axis_index("s")
    lane = lax.iota(jnp.int32, 16)
    cbase = t * CHUNK

    def digit_of(k, sh):
        if sh:
            k = lax.shift_right_logical(k, jnp.int32(sh))
        return lax.bitwise_and(k, jnp.int32(RADIX - 1))

    def gloop(j, _):
        vinit[pl.ds(j * 16, 16)] = cbase + j * 16 + lane
        return 0
    lax.fori_loop(0, VREGS, gloop, 0)

    HALF_A = 24

    def one_pass(sh, src_v, dst_k, dst_v):
        def zloop(z):
            hb[pl.ds(z * 16, 16)] = jnp.zeros((16,), jnp.int32)
        plsc.parallel_loop(0, RADIX // 16, unroll=4)(zloop)

        def hloop(j):
            sl = pl.ds(j * 16, 16)
            d = digit_of(kv[sl], sh)
            cnt, lastm = plsc.scan_count(d)
            dbuf[sl] = d
            cntbuf[sl] = cnt
            lastbuf[sl] = jnp.where(lastm, 1, 0).astype(jnp.int32)
            plsc.addupdate_scatter(hb, [d], cnt, mask=lastm)
        plsc.parallel_loop(0, HALF_A * 8, unroll=4)(hloop)

        def cloop(z):
            hba[pl.ds(z * 16, 16)] = hb[pl.ds(z * 16, 16)]
        plsc.parallel_loop(0, RADIX // 16, unroll=4)(cloop)
        plsc.parallel_loop(HALF_A * 8, VREGS, unroll=4)(hloop)
        pltpu.sync_copy(hb, hist_sp.at[t])
        plsc.subcore_barrier()

        pltpu.sync_copy(hist_sp, hm)

        def sloop(dv, carry):
            sl = pl.ds(dv * 16, 16)
            z16 = jnp.zeros((16,), jnp.int32)

            def tloop(tp, tc):
                tot, part = tc
                hv = hm[tp, sl]
                tot = tot + hv
                part = part + jnp.where(tp < t, hv, z16)
                return tot, part
            tot, part = lax.fori_loop(0, NT, tloop, (z16, z16))
            incl = plsc.cumsum(tot)
            hb[sl] = incl - tot + part + carry
            return carry + jnp.max(incl)
        lax.fori_loop(0, RADIX // 16, sloop, jnp.int32(0))

        def bloop(z, _):
            sl = pl.ds(z * 16, 16)
            hba[sl] = hba[sl] + hb[sl]
            return 0
        lax.fori_loop(0, RADIX // 16, bloop, 0)

        n_dma = 2 if dst_k is not None else 1

        def rank_vreg(j, base_ref):
            sl = pl.ds(j * 16, 16)
            d = dbuf[sl]
            cnt = cntbuf[sl]
            lastm = lastbuf[sl] != 0
            base = plsc.load_gather(base_ref, [d])
            posv[j >> 3, pl.ds((j & 7) * 16, 16)] = base + cnt - 1
            plsc.addupdate_scatter(base_ref, [d], cnt, mask=lastm)

        def fire(s):
            idx = posv.at[s]
            src = pl.ds(s * 128, 128)
            if dst_k is not None:
                pltpu.async_copy(kv.at[src], dst_k.at[idx], sem)
            pltpu.async_copy(src_v.at[src], dst_v.at[idx], sem)

        def rgroup(g, _):
            for q in range(8):
                rank_vreg(g * 8 + q, hb)
                rank_vreg(HALF_A * 8 + g * 8 + q, hba)
            fire(g)
            fire(HALF_A + g)
            return 0
        lax.fori_loop(0, HALF_A, rgroup, 0)
        for q in range(8):
            rank_vreg(2 * HALF_A * 8 + q, hba)
        fire(2 * HALF_A)

        def dloop(s, _):
            pltpu.make_async_copy(keys_hbm.at[pl.ds(0, 128)],
                                  kv.at[pl.ds(0, 128)], sem).wait()
            return 0
        lax.fori_loop(0, SLICES * n_dma, dloop, 0)
        plsc.subcore_barrier()

    def do_row(ri, _):
        row = c * rows_per_core + ri
        hoff = row * NPAD + cbase

        pltpu.sync_copy(keys_hbm.at[pl.ds(hoff, CHUNK)], kv)
        one_pass(0, vinit, ka, va)

        pltpu.sync_copy(ka.at[pl.ds(cbase, CHUNK)], kv)
        pltpu.sync_copy(va.at[pl.ds(cbase, CHUNK)], vv)
        one_pass(8, vv, kb, vb)

        pltpu.sync_copy(kb.at[pl.ds(cbase, CHUNK)], kv)
        pltpu.sync_copy(vb.at[pl.ds(cbase, CHUNK)], vv)
        one_pass(16, vv, ka, va)

        pltpu.sync_copy(ka.at[pl.ds(cbase, CHUNK)], kv)
        pltpu.sync_copy(va.at[pl.ds(cbase, CHUNK)], vv)
        one_pass(24, vv, None, vb)

        pltpu.sync_copy(vb.at[pl.ds(cbase, CHUNK)], out_hbm.at[pl.ds(hoff, CHUNK)])
        plsc.subcore_barrier()
        return 0

    lax.fori_loop(0, rows_per_core, do_row, 0)


@jax.jit
def kernel(sample, filter_bias, ent_embd, rel_embd, wrh, wrt):
    h = jnp.take(ent_embd, sample[:, 0], axis=0)
    r = jnp.take(rel_embd, sample[:, 1], axis=0)
    wh = jnp.take(wrh, sample[:, 1], axis=0)
    wt = jnp.take(wrt, sample[:, 1], axis=0)
    v = wh * h + r

    vt, wtt = v.T, wt.T
    nsplit = 2
    hb2 = B // nsplit
    keys_parts = [
        _score_keys(ent_embd, vt, wtt,
                    filter_bias[p * hb2:(p + 1) * hb2], p * hb2, hb2)
        for p in range(nsplit)
    ]

    mesh = plsc.VectorSubcoreMesh(core_axis_name="c", subcore_axis_name="s")
    radix = functools.partial(
        pl.kernel,
        out_type=jax.ShapeDtypeStruct((hb2 * NPAD,), jnp.int32),
        mesh=mesh,
        compiler_params=pltpu.CompilerParams(needs_layout_passes=False),
        scratch_types=[
            pltpu.VMEM((CHUNK,), jnp.int32),
            pltpu.VMEM((CHUNK,), jnp.int32),
            pltpu.VMEM((CHUNK,), jnp.int32),
            pltpu.VMEM((CHUNK,), jnp.int32),
            pltpu.VMEM((CHUNK,), jnp.int32),
            pltpu.VMEM((CHUNK,), jnp.int32),
            pltpu.VMEM((SLICES, 128), jnp.int32),
            pltpu.VMEM((RADIX,), jnp.int32),
            pltpu.VMEM((RADIX,), jnp.int32),
            pltpu.VMEM((NT, RADIX), jnp.int32),
            pltpu.VMEM_SHARED((NT, RADIX), jnp.int32),
            pltpu.VMEM_SHARED((NPAD,), jnp.int32),
            pltpu.VMEM_SHARED((NPAD,), jnp.int32),
            pltpu.VMEM_SHARED((NPAD,), jnp.int32),
            pltpu.VMEM_SHARED((NPAD,), jnp.int32),
            pltpu.SemaphoreType.DMA,
        ],
    )(functools.partial(_radix_body, rows_per_core=hb2 // NC))
    order = jnp.concatenate(
        [radix(kp.reshape(-1)).reshape(hb2, NPAD) for kp in keys_parts],
        axis=0)
    return order[:, :N]
